# speedup vs baseline: 1.4801x; 1.4801x over previous
import functools

import jax
import jax.numpy as jnp
from jax import lax
from jax.experimental import pallas as pl
from jax.experimental.pallas import tpu as pltpu
from jax.experimental.pallas import tpu_sc as plsc

N_CLASSES = 1000
N_REGIMES = 64
N_EXCIPIENTS = 128
BATCH = 16384
TABLE = N_CLASSES * N_REGIMES * N_EXCIPIENTS

NC = 2
NS = 16
L = 16
NW = NC * NS
BPW = BATCH // NW
CHUNK = 128
NCHUNK = BPW // CHUNK


def _tanh(x):
    a = jnp.abs(x)
    z = jnp.exp(a * -2.0)
    t = (1.0 - z) / (1.0 + z)
    return jnp.where(x < 0, -t, t)


def _sc_body(p_hbm, r_hbm, e_hbm, elo_hbm, ehi_hbm,
             s_hbm, d_hbm, wl_hbm, wh_hbm,
             o_score, o_d, o_wl, o_wh, o_eh, o_base, o_conc, o_res,
             pv, rv, ev, lov, hiv, lin,
             gs, gd, gwl, gwh,
             cd, ceh, cbase, cconc, cres,
             sem):
    wid = lax.axis_index("s") * NC + lax.axis_index("c")
    base = wid * BPW

    pltpu.sync_copy(p_hbm.at[pl.ds(base, BPW)], pv)
    pltpu.sync_copy(r_hbm.at[pl.ds(base, BPW)], rv)
    pltpu.sync_copy(e_hbm.at[pl.ds(base, BPW)], ev)
    pltpu.sync_copy(elo_hbm.at[pl.ds(base, BPW)], lov)
    pltpu.sync_copy(ehi_hbm.at[pl.ds(base, BPW)], hiv)

    for i in range(BPW // L):
        sl = pl.ds(i * L, L)
        lin[sl] = (pv[sl] * (N_REGIMES * N_EXCIPIENTS)
                   + rv[sl] * N_EXCIPIENTS + ev[sl])

    copies = []
    for tab, dst in ((s_hbm, gs), (d_hbm, gd), (wl_hbm, gwl), (wh_hbm, gwh)):
        for j in range(NCHUNK):
            sl = pl.ds(j * CHUNK, CHUNK)
            copies.append(
                pltpu.async_copy(tab.at[lin.at[sl]], dst.at[sl], sem))
    for c in copies:
        c.wait()

    for i in range(BPW // L):
        sl = pl.ds(i * L, L)
        dd = jnp.clip(gd[sl], -2.0, 2.0)
        eh = _tanh(hiv[sl])
        bt = gs[sl] + dd
        ct = gwl[sl] * lov[sl] + gwh[sl] * eh
        cd[sl] = dd
        ceh[sl] = eh
        cbase[sl] = bt
        cconc[sl] = ct
        cres[sl] = bt * ct

    out_sl = pl.ds(base, BPW)
    pltpu.sync_copy(gs, o_score.at[out_sl])
    pltpu.sync_copy(cd, o_d.at[out_sl])
    pltpu.sync_copy(gwl, o_wl.at[out_sl])
    pltpu.sync_copy(gwh, o_wh.at[out_sl])
    pltpu.sync_copy(ceh, o_eh.at[out_sl])
    pltpu.sync_copy(cbase, o_base.at[out_sl])
    pltpu.sync_copy(cconc, o_conc.at[out_sl])
    pltpu.sync_copy(cres, o_res.at[out_sl])


@jax.jit
def _sc_gather(p_idx, r_idx, e_idx, e_low, e_high, s, d, wl, wh):
    f32 = jnp.float32
    out_type = tuple(jax.ShapeDtypeStruct((BATCH,), f32) for _ in range(8))
    scratch = [
        pltpu.VMEM((BPW,), jnp.int32),
        pltpu.VMEM((BPW,), jnp.int32),
        pltpu.VMEM((BPW,), jnp.int32),
        pltpu.VMEM((BPW,), f32),
        pltpu.VMEM((BPW,), f32),
        pltpu.VMEM((BPW,), jnp.int32),
        pltpu.VMEM((BPW,), f32),
        pltpu.VMEM((BPW,), f32),
        pltpu.VMEM((BPW,), f32),
        pltpu.VMEM((BPW,), f32),
        pltpu.VMEM((BPW,), f32),
        pltpu.VMEM((BPW,), f32),
        pltpu.VMEM((BPW,), f32),
        pltpu.VMEM((BPW,), f32),
        pltpu.VMEM((BPW,), f32),
        pltpu.SemaphoreType.DMA,
    ]
    run = pl.kernel(
        _sc_body,
        out_type=out_type,
        mesh=plsc.VectorSubcoreMesh(core_axis_name="c", subcore_axis_name="s"),
        scratch_types=scratch,
    )
    return run(p_idx, r_idx, e_idx, e_low, e_high, s, d, wl, wh)


def kernel(p_idx, r_idx, e_idx, e_low_norm, e_high_norm,
           static_scores, delta, w_L, w_H):
    el = e_low_norm.reshape(-1)
    score, dd, wl, wh, eh, base_term, conc_term, result = _sc_gather(
        p_idx.astype(jnp.int32), r_idx.astype(jnp.int32),
        e_idx.astype(jnp.int32),
        el, e_high_norm.reshape(-1),
        static_scores.reshape(-1), delta.reshape(-1),
        w_L.reshape(-1), w_H.reshape(-1))
    details = {
        'static_score': score,
        'delta': dd,
        'w_L': wl,
        'w_H': wh,
        'e_low_norm': el,
        'e_high_norm': eh,
        'base_term': base_term,
        'conc_term': conc_term,
        'result': result,
    }
    return (result[:, None], details)


# speedup vs baseline: 1.7052x; 1.1521x over previous
import jax
import jax.numpy as jnp
from jax import lax
from jax.experimental import pallas as pl
from jax.experimental.pallas import tpu as pltpu
from jax.experimental.pallas import tpu_sc as plsc

N_CLASSES = 1000
N_REGIMES = 64
N_EXCIPIENTS = 128
BATCH = 16384

NC = 2
NS = 16
L = 16
NW = NC * NS
BPW = BATCH // NW
CHUNK = 128
NCHUNK = BPW // CHUNK


def _tanh(x):
    a = jnp.abs(x)
    z = jnp.exp(a * -2.0)
    t = (1.0 - z) / (1.0 + z)
    return jnp.where(x < 0, -t, t)


def _sc_body(p_hbm, r_hbm, e_hbm, elo_hbm, ehi_hbm,
             s_hbm, d_hbm, wl_hbm, wh_hbm,
             o_score, o_d, o_wl, o_wh, o_eh, o_base, o_conc, o_res,
             o_el, o_res2,
             pv, rv, ev, lov, hiv, lin,
             gs, gd, gwl, gwh,
             cd, ceh, cbase, cconc, cres,
             sem_in, sem_g, sem_o):
    wid = lax.axis_index("s") * NC + lax.axis_index("c")
    base = wid * BPW
    in_sl = pl.ds(base, BPW)

    loads = [pltpu.async_copy(p_hbm.at[in_sl], pv, sem_in),
             pltpu.async_copy(r_hbm.at[in_sl], rv, sem_in),
             pltpu.async_copy(e_hbm.at[in_sl], ev, sem_in),
             pltpu.async_copy(elo_hbm.at[in_sl], lov, sem_in),
             pltpu.async_copy(ehi_hbm.at[in_sl], hiv, sem_in)]
    for c in loads[:3]:
        c.wait()

    for i in range(BPW // L):
        sl = pl.ds(i * L, L)
        lin[sl] = (pv[sl] * (N_REGIMES * N_EXCIPIENTS)
                   + rv[sl] * N_EXCIPIENTS + ev[sl])

    gathers = []
    for tab, dst in ((s_hbm, gs), (d_hbm, gd), (wl_hbm, gwl), (wh_hbm, gwh)):
        for j in range(NCHUNK):
            sl = pl.ds(j * CHUNK, CHUNK)
            gathers.append(
                pltpu.async_copy(tab.at[lin.at[sl]], dst.at[sl], sem_g))

    loads[3].wait()
    loads[4].wait()

    for i in range(BPW // L):
        sl = pl.ds(i * L, L)
        ceh[sl] = _tanh(hiv[sl])

    stores = [pltpu.async_copy(ceh, o_eh.at[in_sl], sem_o),
              pltpu.async_copy(lov, o_el.at[in_sl], sem_o)]

    for c in gathers:
        c.wait()

    stores += [pltpu.async_copy(gs, o_score.at[in_sl], sem_o),
               pltpu.async_copy(gwl, o_wl.at[in_sl], sem_o),
               pltpu.async_copy(gwh, o_wh.at[in_sl], sem_o)]

    for i in range(BPW // L):
        sl = pl.ds(i * L, L)
        dd = jnp.clip(gd[sl], -2.0, 2.0)
        bt = gs[sl] + dd
        ct = gwl[sl] * lov[sl] + gwh[sl] * ceh[sl]
        cd[sl] = dd
        cbase[sl] = bt
        cconc[sl] = ct
        cres[sl] = bt * ct

    stores += [pltpu.async_copy(cd, o_d.at[in_sl], sem_o),
               pltpu.async_copy(cbase, o_base.at[in_sl], sem_o),
               pltpu.async_copy(cconc, o_conc.at[in_sl], sem_o),
               pltpu.async_copy(cres, o_res.at[in_sl], sem_o),
               pltpu.async_copy(cres, o_res2.at[0, in_sl], sem_o)]
    for c in stores:
        c.wait()


@jax.jit
def _sc_gather(p_idx, r_idx, e_idx, e_low, e_high, s, d, wl, wh):
    f32 = jnp.float32
    out_type = tuple(jax.ShapeDtypeStruct((BATCH,), f32) for _ in range(9)) \
        + (jax.ShapeDtypeStruct((1, BATCH), f32),)
    scratch = [
        pltpu.VMEM((BPW,), jnp.int32),
        pltpu.VMEM((BPW,), jnp.int32),
        pltpu.VMEM((BPW,), jnp.int32),
        pltpu.VMEM((BPW,), f32),
        pltpu.VMEM((BPW,), f32),
        pltpu.VMEM((BPW,), jnp.int32),
        pltpu.VMEM((BPW,), f32),
        pltpu.VMEM((BPW,), f32),
        pltpu.VMEM((BPW,), f32),
        pltpu.VMEM((BPW,), f32),
        pltpu.VMEM((BPW,), f32),
        pltpu.VMEM((BPW,), f32),
        pltpu.VMEM((BPW,), f32),
        pltpu.VMEM((BPW,), f32),
        pltpu.VMEM((BPW,), f32),
        pltpu.SemaphoreType.DMA,
        pltpu.SemaphoreType.DMA,
        pltpu.SemaphoreType.DMA,
    ]
    run = pl.kernel(
        _sc_body,
        out_type=out_type,
        mesh=plsc.VectorSubcoreMesh(core_axis_name="c", subcore_axis_name="s"),
        scratch_types=scratch,
    )
    return run(p_idx, r_idx, e_idx, e_low, e_high, s, d, wl, wh)


def kernel(p_idx, r_idx, e_idx, e_low_norm, e_high_norm,
           static_scores, delta, w_L, w_H):
    (score, dd, wl, wh, eh, base_term, conc_term, result, el,
     result2d) = _sc_gather(
        p_idx.astype(jnp.int32), r_idx.astype(jnp.int32),
        e_idx.astype(jnp.int32),
        e_low_norm.reshape(-1), e_high_norm.reshape(-1),
        static_scores.reshape(-1), delta.reshape(-1),
        w_L.reshape(-1), w_H.reshape(-1))
    details = {
        'static_score': score,
        'delta': dd,
        'w_L': wl,
        'w_H': wh,
        'e_low_norm': el,
        'e_high_norm': eh,
        'base_term': base_term,
        'conc_term': conc_term,
        'result': result,
    }
    return (result2d.reshape(BATCH, 1), details)


# speedup vs baseline: 1.7631x; 1.0340x over previous
import jax
import jax.numpy as jnp
from jax import lax
from jax.experimental import pallas as pl
from jax.experimental.pallas import tpu as pltpu
from jax.experimental.pallas import tpu_sc as plsc

N_CLASSES = 1000
N_REGIMES = 64
N_EXCIPIENTS = 128
BATCH = 16384

NC = 2
NS = 16
L = 16
NW = NC * NS
BPW = BATCH // NW
CHUNK = 128
NCHUNK = BPW // CHUNK


def _tanh(x):
    a = jnp.abs(x)
    z = jnp.exp(a * -2.0)
    t = (1.0 - z) / (1.0 + z)
    return jnp.where(x < 0, -t, t)


def _sc_body(p_hbm, r_hbm, e_hbm, elo_hbm, ehi_hbm,
             s_hbm, d_hbm, wl_hbm, wh_hbm,
             o_score, o_d, o_wl, o_wh, o_eh, o_base, o_conc, o_res,
             o_el, o_res2,
             pv, rv, ev, lov, hiv, lin,
             gs, gd, gwl, gwh,
             cd, ceh, cbase, cconc, cres,
             sem_in, sem_g, sem_o):
    wid = lax.axis_index("s") * NC + lax.axis_index("c")
    base = wid * BPW
    in_sl = pl.ds(base, BPW)

    loads = [pltpu.async_copy(p_hbm.at[in_sl], pv, sem_in),
             pltpu.async_copy(r_hbm.at[in_sl], rv, sem_in),
             pltpu.async_copy(e_hbm.at[in_sl], ev, sem_in),
             pltpu.async_copy(elo_hbm.at[in_sl], lov, sem_in),
             pltpu.async_copy(ehi_hbm.at[in_sl], hiv, sem_in)]
    for c in loads[:3]:
        c.wait()

    gathers = []
    for j in range(NCHUNK):
        for i in range(j * (CHUNK // L), (j + 1) * (CHUNK // L)):
            sl = pl.ds(i * L, L)
            lin[sl] = (pv[sl] * (N_REGIMES * N_EXCIPIENTS)
                       + rv[sl] * N_EXCIPIENTS + ev[sl])
        csl = pl.ds(j * CHUNK, CHUNK)
        for tab, dst in ((s_hbm, gs), (d_hbm, gd),
                         (wl_hbm, gwl), (wh_hbm, gwh)):
            gathers.append(
                pltpu.async_copy(tab.at[lin.at[csl]], dst.at[csl], sem_g))

    loads[3].wait()
    loads[4].wait()

    def eh_body(i, _):
        sl = pl.ds(i * L, L)
        ceh[sl] = _tanh(hiv[sl])
        return 0
    lax.fori_loop(0, BPW // L, eh_body, 0)

    stores = [pltpu.async_copy(ceh, o_eh.at[in_sl], sem_o),
              pltpu.async_copy(lov, o_el.at[in_sl], sem_o)]

    for c in gathers:
        c.wait()

    stores += [pltpu.async_copy(gs, o_score.at[in_sl], sem_o),
               pltpu.async_copy(gwl, o_wl.at[in_sl], sem_o),
               pltpu.async_copy(gwh, o_wh.at[in_sl], sem_o)]

    def combine_body(i, _):
        sl = pl.ds(i * L, L)
        dd = jnp.clip(gd[sl], -2.0, 2.0)
        bt = gs[sl] + dd
        ct = gwl[sl] * lov[sl] + gwh[sl] * ceh[sl]
        cd[sl] = dd
        cbase[sl] = bt
        cconc[sl] = ct
        cres[sl] = bt * ct
        return 0
    lax.fori_loop(0, BPW // L, combine_body, 0)

    stores += [pltpu.async_copy(cd, o_d.at[in_sl], sem_o),
               pltpu.async_copy(cbase, o_base.at[in_sl], sem_o),
               pltpu.async_copy(cconc, o_conc.at[in_sl], sem_o),
               pltpu.async_copy(cres, o_res.at[in_sl], sem_o),
               pltpu.async_copy(cres, o_res2.at[0, in_sl], sem_o)]
    for c in stores:
        c.wait()


@jax.jit
def _sc_gather(p_idx, r_idx, e_idx, e_low, e_high, s, d, wl, wh):
    f32 = jnp.float32
    out_type = tuple(jax.ShapeDtypeStruct((BATCH,), f32) for _ in range(9)) \
        + (jax.ShapeDtypeStruct((1, BATCH), f32),)
    scratch = [
        pltpu.VMEM((BPW,), jnp.int32),
        pltpu.VMEM((BPW,), jnp.int32),
        pltpu.VMEM((BPW,), jnp.int32),
        pltpu.VMEM((BPW,), f32),
        pltpu.VMEM((BPW,), f32),
        pltpu.VMEM((BPW,), jnp.int32),
        pltpu.VMEM((BPW,), f32),
        pltpu.VMEM((BPW,), f32),
        pltpu.VMEM((BPW,), f32),
        pltpu.VMEM((BPW,), f32),
        pltpu.VMEM((BPW,), f32),
        pltpu.VMEM((BPW,), f32),
        pltpu.VMEM((BPW,), f32),
        pltpu.VMEM((BPW,), f32),
        pltpu.VMEM((BPW,), f32),
        pltpu.SemaphoreType.DMA,
        pltpu.SemaphoreType.DMA,
        pltpu.SemaphoreType.DMA,
    ]
    run = pl.kernel(
        _sc_body,
        out_type=out_type,
        mesh=plsc.VectorSubcoreMesh(core_axis_name="c", subcore_axis_name="s"),
        scratch_types=scratch,
    )
    return run(p_idx, r_idx, e_idx, e_low, e_high, s, d, wl, wh)


def kernel(p_idx, r_idx, e_idx, e_low_norm, e_high_norm,
           static_scores, delta, w_L, w_H):
    (score, dd, wl, wh, eh, base_term, conc_term, result, el,
     result2d) = _sc_gather(
        p_idx.astype(jnp.int32), r_idx.astype(jnp.int32),
        e_idx.astype(jnp.int32),
        e_low_norm.reshape(-1), e_high_norm.reshape(-1),
        static_scores.reshape(-1), delta.reshape(-1),
        w_L.reshape(-1), w_H.reshape(-1))
    details = {
        'static_score': score,
        'delta': dd,
        'w_L': wl,
        'w_H': wh,
        'e_low_norm': el,
        'e_high_norm': eh,
        'base_term': base_term,
        'conc_term': conc_term,
        'result': result,
    }
    return (result2d.reshape(BATCH, 1), details)


# speedup vs baseline: 1.8043x; 1.0233x over previous
import jax
import jax.numpy as jnp
from jax import lax
from jax.experimental import pallas as pl
from jax.experimental.pallas import tpu as pltpu
from jax.experimental.pallas import tpu_sc as plsc

N_CLASSES = 1000
N_REGIMES = 64
N_EXCIPIENTS = 128
BATCH = 16384

NC = 2
NS = 16
L = 16
NW = NC * NS
BPW = BATCH // NW
CHUNK = 512
NCHUNK = BPW // CHUNK


def _tanh(x):
    a = jnp.abs(x)
    z = jnp.exp(a * -2.0)
    t = (1.0 - z) / (1.0 + z)
    return jnp.where(x < 0, -t, t)


def _sc_body(p_hbm, r_hbm, e_hbm, elo_hbm, ehi_hbm,
             s_hbm, d_hbm, wl_hbm, wh_hbm,
             o_score, o_d, o_wl, o_wh, o_eh, o_base, o_conc, o_res,
             o_el, o_res2,
             pv, rv, ev, lov, hiv, lin,
             gs, gd, gwl, gwh,
             cd, ceh, cbase, cconc, cres,
             sem_in, sem_g, sem_o):
    wid = lax.axis_index("s") * NC + lax.axis_index("c")
    base = wid * BPW
    in_sl = pl.ds(base, BPW)

    loads = [pltpu.async_copy(p_hbm.at[in_sl], pv, sem_in),
             pltpu.async_copy(r_hbm.at[in_sl], rv, sem_in),
             pltpu.async_copy(e_hbm.at[in_sl], ev, sem_in),
             pltpu.async_copy(elo_hbm.at[in_sl], lov, sem_in),
             pltpu.async_copy(ehi_hbm.at[in_sl], hiv, sem_in)]
    for c in loads[:3]:
        c.wait()

    def lin_body(i, _):
        sl = pl.ds(i * L, L)
        lin[sl] = (pv[sl] * (N_REGIMES * N_EXCIPIENTS)
                   + rv[sl] * N_EXCIPIENTS + ev[sl])
        return 0
    lax.fori_loop(0, BPW // L, lin_body, 0)

    gathers = []
    for j in range(NCHUNK):
        csl = pl.ds(j * CHUNK, CHUNK)
        for tab, dst in ((s_hbm, gs), (d_hbm, gd),
                         (wl_hbm, gwl), (wh_hbm, gwh)):
            gathers.append(
                pltpu.async_copy(tab.at[lin.at[csl]], dst.at[csl], sem_g))

    loads[3].wait()
    loads[4].wait()

    def eh_body(i, _):
        sl = pl.ds(i * L, L)
        ceh[sl] = _tanh(hiv[sl])
        return 0
    lax.fori_loop(0, BPW // L, eh_body, 0)

    stores = [pltpu.async_copy(ceh, o_eh.at[in_sl], sem_o),
              pltpu.async_copy(lov, o_el.at[in_sl], sem_o)]

    for c in gathers:
        c.wait()

    stores += [pltpu.async_copy(gs, o_score.at[in_sl], sem_o),
               pltpu.async_copy(gwl, o_wl.at[in_sl], sem_o),
               pltpu.async_copy(gwh, o_wh.at[in_sl], sem_o)]

    def combine_body(i, _):
        sl = pl.ds(i * L, L)
        dd = jnp.clip(gd[sl], -2.0, 2.0)
        bt = gs[sl] + dd
        ct = gwl[sl] * lov[sl] + gwh[sl] * ceh[sl]
        cd[sl] = dd
        cbase[sl] = bt
        cconc[sl] = ct
        cres[sl] = bt * ct
        return 0
    lax.fori_loop(0, BPW // L, combine_body, 0)

    stores += [pltpu.async_copy(cd, o_d.at[in_sl], sem_o),
               pltpu.async_copy(cbase, o_base.at[in_sl], sem_o),
               pltpu.async_copy(cconc, o_conc.at[in_sl], sem_o),
               pltpu.async_copy(cres, o_res.at[in_sl], sem_o),
               pltpu.async_copy(cres, o_res2.at[0, in_sl], sem_o)]
    for c in stores:
        c.wait()


@jax.jit
def _sc_gather(p_idx, r_idx, e_idx, e_low, e_high, s, d, wl, wh):
    f32 = jnp.float32
    out_type = tuple(jax.ShapeDtypeStruct((BATCH,), f32) for _ in range(9)) \
        + (jax.ShapeDtypeStruct((1, BATCH), f32),)
    scratch = [
        pltpu.VMEM((BPW,), jnp.int32),
        pltpu.VMEM((BPW,), jnp.int32),
        pltpu.VMEM((BPW,), jnp.int32),
        pltpu.VMEM((BPW,), f32),
        pltpu.VMEM((BPW,), f32),
        pltpu.VMEM((BPW,), jnp.int32),
        pltpu.VMEM((BPW,), f32),
        pltpu.VMEM((BPW,), f32),
        pltpu.VMEM((BPW,), f32),
        pltpu.VMEM((BPW,), f32),
        pltpu.VMEM((BPW,), f32),
        pltpu.VMEM((BPW,), f32),
        pltpu.VMEM((BPW,), f32),
        pltpu.VMEM((BPW,), f32),
        pltpu.VMEM((BPW,), f32),
        pltpu.SemaphoreType.DMA,
        pltpu.SemaphoreType.DMA,
        pltpu.SemaphoreType.DMA,
    ]
    run = pl.kernel(
        _sc_body,
        out_type=out_type,
        mesh=plsc.VectorSubcoreMesh(core_axis_name="c", subcore_axis_name="s"),
        scratch_types=scratch,
    )
    return run(p_idx, r_idx, e_idx, e_low, e_high, s, d, wl, wh)


def kernel(p_idx, r_idx, e_idx, e_low_norm, e_high_norm,
           static_scores, delta, w_L, w_H):
    (score, dd, wl, wh, eh, base_term, conc_term, result, el,
     result2d) = _sc_gather(
        p_idx.astype(jnp.int32), r_idx.astype(jnp.int32),
        e_idx.astype(jnp.int32),
        e_low_norm.reshape(-1), e_high_norm.reshape(-1),
        static_scores.reshape(-1), delta.reshape(-1),
        w_L.reshape(-1), w_H.reshape(-1))
    details = {
        'static_score': score,
        'delta': dd,
        'w_L': wl,
        'w_H': wh,
        'e_low_norm': el,
        'e_high_norm': eh,
        'base_term': base_term,
        'conc_term': conc_term,
        'result': result,
    }
    return (result2d.reshape(BATCH, 1), details)
